# Initial kernel scaffold; baseline (speedup 1.0000x reference)
import jax
import jax.numpy as jnp
from jax.experimental import pallas as pl


def kernel(user_id, pos_id, neg_ids, user_emb, centroids_0, centroids_1, item_mu, item_logvar):
    raise NotImplementedError("write your pallas kernel here")


# speedup vs baseline: 2.5936x; 2.5936x over previous
import functools

import jax
import jax.numpy as jnp
from jax import lax
from jax.experimental import pallas as pl
from jax.experimental.pallas import tpu as pltpu
from jax.experimental.pallas import tpu_sc as plsc

_B = 4096
_L = 50
_D = 64
_NCEN = 32
_CD = 32
_NITEM_ROWS = _B * 2 * _L

_NC = 2
_NS = 16
_NW = _NC * _NS
_CH = 128
_ITEM_PER_W = _NITEM_ROWS // _NW
_UCH = _B // _NW


def _sc_gather(ids, uid, item_mu, item_logvar, user_emb):
    mesh = plsc.VectorSubcoreMesh(core_axis_name="c", subcore_axis_name="s")

    @functools.partial(
        pl.kernel,
        out_type=(
            jax.ShapeDtypeStruct((_NITEM_ROWS, _D), jnp.float32),
            jax.ShapeDtypeStruct((_NITEM_ROWS, _D), jnp.float32),
            jax.ShapeDtypeStruct((_B, _D), jnp.float32),
        ),
        mesh=mesh,
        compiler_params=pltpu.CompilerParams(use_tc_tiling_on_sc=False),
        scratch_types=[
            pltpu.VMEM((_CH,), jnp.int32),
            pltpu.VMEM((_CH, _D), jnp.float32),
            pltpu.VMEM((_CH, _D), jnp.float32),
            pltpu.SemaphoreType.DMA,
            pltpu.SemaphoreType.DMA,
        ],
    )
    def k(ids_hbm, uid_hbm, mu_hbm, lv_hbm, ue_hbm,
          out_mu, out_lv, out_ue, idx_v, a_v, b_v, sem1, sem2):
        wid = lax.axis_index("s") * _NC + lax.axis_index("c")

        ubase = wid * _UCH
        pltpu.sync_copy(uid_hbm.at[pl.ds(ubase, _UCH)], idx_v)
        pltpu.async_copy(ue_hbm.at[idx_v], a_v, sem1).wait()
        pltpu.sync_copy(a_v, out_ue.at[pl.ds(ubase, _UCH)])

        nchunks = _ITEM_PER_W // _CH

        def body(c, carry):
            base = wid * _ITEM_PER_W + c * _CH
            pltpu.sync_copy(ids_hbm.at[pl.ds(base, _CH)], idx_v)
            cp1 = pltpu.async_copy(mu_hbm.at[idx_v], a_v, sem1)
            cp2 = pltpu.async_copy(lv_hbm.at[idx_v], b_v, sem2)
            cp1.wait()
            cp2.wait()
            pltpu.sync_copy(a_v, out_mu.at[pl.ds(base, _CH)])
            pltpu.sync_copy(b_v, out_lv.at[pl.ds(base, _CH)])
            return carry

        lax.fori_loop(0, nchunks, body, 0)

    return k(ids, uid, item_mu, item_logvar, user_emb)


_BU = 128


def _tc_body(ue_r, g0_r, g1_r, c0_r, c1_r, mu_r, lv_r, ep_p_r, ep_n_r,
             pos_o, neg_o):
    ue = ue_r[...]
    iota = lax.broadcasted_iota(jnp.int32, (_BU, _NCEN), 1)
    parts = []
    for p, (c_r, g_r) in enumerate(((c0_r, g0_r), (c1_r, g1_r))):
        sub = ue[:, p * _CD:(p + 1) * _CD]
        c = c_r[...]
        un = jnp.sum(sub * sub, axis=1, keepdims=True)
        cn = jnp.sum(c * c, axis=1)[None, :]
        dot = lax.dot_general(sub, c, (((1,), (1,)), ((), ())),
                              preferred_element_type=jnp.float32)
        dist = -(un + cn - 2.0 * dot)
        scored = dist + g_r[...]
        m = jnp.max(scored, axis=1, keepdims=True)
        cand = jnp.where(scored == m, iota, _NCEN)
        idx = jnp.min(cand, axis=1)
        oh = (iota == idx[:, None]).astype(jnp.float32)
        parts.append(jnp.sum(oh[:, :, None] * c[None, :, :], axis=1))
    uv = jnp.concatenate(parts, axis=1)

    std = jnp.exp(0.5 * lv_r[...])
    mu = mu_r[...]
    items_p = ep_p_r[...] * std[:, :_L, :] + mu[:, :_L, :]
    items_n = ep_n_r[...] * std[:, _L:, :] + mu[:, _L:, :]
    pos_o[...] = jnp.sum(uv[:, None, :] * items_p, axis=-1)
    neg_o[...] = jnp.sum(uv[:, None, :] * items_n, axis=-1)


def _tc_score(ue, g0, g1, c0, c1, mu3, lv3, ep_p, ep_n):
    grid = (_B // _BU,)
    return pl.pallas_call(
        _tc_body,
        grid=grid,
        in_specs=[
            pl.BlockSpec((_BU, _D), lambda i: (i, 0)),
            pl.BlockSpec((_BU, _NCEN), lambda i: (i, 0)),
            pl.BlockSpec((_BU, _NCEN), lambda i: (i, 0)),
            pl.BlockSpec((_NCEN, _CD), lambda i: (0, 0)),
            pl.BlockSpec((_NCEN, _CD), lambda i: (0, 0)),
            pl.BlockSpec((_BU, 2 * _L, _D), lambda i: (i, 0, 0)),
            pl.BlockSpec((_BU, 2 * _L, _D), lambda i: (i, 0, 0)),
            pl.BlockSpec((_BU, _L, _D), lambda i: (i, 0, 0)),
            pl.BlockSpec((_BU, _L, _D), lambda i: (i, 0, 0)),
        ],
        out_specs=[
            pl.BlockSpec((_BU, _L), lambda i: (i, 0)),
            pl.BlockSpec((_BU, _L), lambda i: (i, 0)),
        ],
        out_shape=[
            jax.ShapeDtypeStruct((_B, _L), jnp.float32),
            jax.ShapeDtypeStruct((_B, _L), jnp.float32),
        ],
    )(ue, g0, g1, c0, c1, mu3, lv3, ep_p, ep_n)


def kernel(user_id, pos_id, neg_ids, user_emb, centroids_0, centroids_1,
           item_mu, item_logvar):
    key = jax.random.key(42)
    gs = []
    for i in range(2):
        u = jax.random.uniform(jax.random.fold_in(key, i), (_B, _NCEN),
                               minval=1e-10, maxval=1.0)
        gs.append(-jnp.log(-jnp.log(u)))
    ep_p = jax.random.normal(jax.random.fold_in(key, 100), (_B, _L, _D),
                             dtype=jnp.float32)
    ep_n = jax.random.normal(jax.random.fold_in(key, 101), (_B, _L, _D),
                             dtype=jnp.float32)

    ids = jnp.concatenate([pos_id.astype(jnp.int32),
                           neg_ids.astype(jnp.int32)], axis=1).reshape(-1)
    uid = user_id.astype(jnp.int32)

    mu_rows, lv_rows, ue_rows = _sc_gather(ids, uid, item_mu, item_logvar,
                                           user_emb)
    mu3 = mu_rows.reshape(_B, 2 * _L, _D)
    lv3 = lv_rows.reshape(_B, 2 * _L, _D)
    pos_s, neg_s = _tc_score(ue_rows, gs[0], gs[1], centroids_0, centroids_1,
                             mu3, lv3, ep_p, ep_n)
    return (pos_s, neg_s)
